# baseline (speedup 1.0000x reference)
import jax
import jax.numpy as jnp
from jax import lax
from jax.experimental import pallas as pl
from jax.experimental.pallas import tpu as pltpu


def _round_up(a: int, b: int) -> int:
    return ((a + b - 1) // b) * b


def _linear_kernel(x_ref, w_ref, b_ref, o_ref):
    acc = lax.dot_general(
        x_ref[...], w_ref[...],
        dimension_numbers=(((1,), (1,)), ((), ())),
        preferred_element_type=jnp.float32,
    )
    o_ref[...] = (acc + b_ref[...]).astype(o_ref.dtype)


def kernel(x, weight, bias):
    N, D = x.shape
    C, D_w = weight.shape
    assert D == D_w
    out_dtype = x.dtype

    TN = 512
    n_pad = _round_up(max(N, 8), TN)
    if n_pad != N:
        x = jnp.pad(x, ((0, n_pad - N), (0, 0)))
    x = x.astype(jnp.bfloat16)

    wt = weight.astype(jnp.bfloat16)
    b2 = bias.reshape(1, C).astype(jnp.float32)

    ni = n_pad // TN
    out = pl.pallas_call(
        _linear_kernel,
        out_shape=jax.ShapeDtypeStruct((n_pad, C), out_dtype),
        grid=(ni,),
        in_specs=[
            pl.BlockSpec((TN, D), lambda i: (i, 0)),
            pl.BlockSpec((C, D), lambda i: (0, 0)),
            pl.BlockSpec((1, C), lambda i: (0, 0)),
        ],
        out_specs=pl.BlockSpec((TN, C), lambda i: (i, 0)),
        compiler_params=pltpu.CompilerParams(
            dimension_semantics=("parallel",)),
    )(x, wt, b2)

    return out[:N]


# speedup vs baseline: 1.5136x; 1.5136x over previous
import jax
import jax.numpy as jnp
from jax import lax
from jax.experimental import pallas as pl
from jax.experimental.pallas import tpu as pltpu


def _round_up(a: int, b: int) -> int:
    return ((a + b - 1) // b) * b


def _linear_kernel(x_ref, w_ref, b_ref, o_ref):
    x = x_ref[...].astype(jnp.bfloat16)
    acc = lax.dot_general(
        x, w_ref[...],
        dimension_numbers=(((1,), (1,)), ((), ())),
        preferred_element_type=jnp.float32,
    )
    o_ref[...] = (acc + b_ref[...]).astype(o_ref.dtype)


def kernel(x, weight, bias):
    N, D = x.shape
    C, D_w = weight.shape
    assert D == D_w
    out_dtype = x.dtype

    TN = 1024
    n_pad = _round_up(max(N, 8), TN)
    if n_pad != N:
        x = jnp.pad(x, ((0, n_pad - N), (0, 0)))

    wt = weight.astype(jnp.bfloat16)
    b2 = bias.reshape(1, C).astype(jnp.float32)

    ni = n_pad // TN
    out = pl.pallas_call(
        _linear_kernel,
        out_shape=jax.ShapeDtypeStruct((n_pad, C), out_dtype),
        grid=(ni,),
        in_specs=[
            pl.BlockSpec((TN, D), lambda i: (i, 0)),
            pl.BlockSpec((C, D), lambda i: (0, 0)),
            pl.BlockSpec((1, C), lambda i: (0, 0)),
        ],
        out_specs=pl.BlockSpec((TN, C), lambda i: (i, 0)),
        compiler_params=pltpu.CompilerParams(
            dimension_semantics=("parallel",)),
    )(x, wt, b2)

    return out[:N]


# speedup vs baseline: 1.5985x; 1.0561x over previous
import jax
import jax.numpy as jnp
from jax import lax
from jax.experimental import pallas as pl
from jax.experimental.pallas import tpu as pltpu


def _round_up(a: int, b: int) -> int:
    return ((a + b - 1) // b) * b


def _linear_kernel(x_ref, w_ref, b_ref, o_ref):
    acc = lax.dot_general(
        x_ref[...], w_ref[...],
        dimension_numbers=(((1,), (1,)), ((), ())),
        preferred_element_type=jnp.float32,
    )
    o_ref[...] = (acc + b_ref[...]).astype(o_ref.dtype)


def kernel(x, weight, bias):
    N, D = x.shape
    C, D_w = weight.shape
    assert D == D_w
    out_dtype = x.dtype

    TN = 1024
    n_pad = _round_up(max(N, 8), TN)
    if n_pad != N:
        x = jnp.pad(x, ((0, n_pad - N), (0, 0)))

    b2 = bias.reshape(1, C).astype(jnp.float32)

    ni = n_pad // TN
    out = pl.pallas_call(
        _linear_kernel,
        out_shape=jax.ShapeDtypeStruct((n_pad, C), out_dtype),
        grid=(ni,),
        in_specs=[
            pl.BlockSpec((TN, D), lambda i: (i, 0)),
            pl.BlockSpec((C, D), lambda i: (0, 0)),
            pl.BlockSpec((1, C), lambda i: (0, 0)),
        ],
        out_specs=pl.BlockSpec((TN, C), lambda i: (i, 0)),
        compiler_params=pltpu.CompilerParams(
            dimension_semantics=("parallel",)),
    )(x, weight, b2)

    return out[:N]


# speedup vs baseline: 1.5992x; 1.0005x over previous
import jax
import jax.numpy as jnp
from jax import lax
from jax.experimental import pallas as pl
from jax.experimental.pallas import tpu as pltpu


def _round_up(a: int, b: int) -> int:
    return ((a + b - 1) // b) * b


def _linear_kernel(x_ref, w_ref, b_ref, o_ref):
    acc = lax.dot_general(
        x_ref[...], w_ref[...],
        dimension_numbers=(((1,), (1,)), ((), ())),
        preferred_element_type=jnp.float32,
    )
    o_ref[...] = (acc + b_ref[...]).astype(o_ref.dtype)


def kernel(x, weight, bias):
    N, D = x.shape
    C, D_w = weight.shape
    assert D == D_w
    out_dtype = x.dtype

    TN = 512
    n_pad = _round_up(max(N, 8), TN)
    if n_pad != N:
        x = jnp.pad(x, ((0, n_pad - N), (0, 0)))

    b2 = bias.reshape(1, C).astype(jnp.float32)

    ni = n_pad // TN
    out = pl.pallas_call(
        _linear_kernel,
        out_shape=jax.ShapeDtypeStruct((n_pad, C), out_dtype),
        grid=(ni,),
        in_specs=[
            pl.BlockSpec((TN, D), lambda i: (i, 0)),
            pl.BlockSpec((C, D), lambda i: (0, 0)),
            pl.BlockSpec((1, C), lambda i: (0, 0)),
        ],
        out_specs=pl.BlockSpec((TN, C), lambda i: (i, 0)),
        compiler_params=pltpu.CompilerParams(
            dimension_semantics=("parallel",)),
    )(x, weight, b2)

    return out[:N]
